# baseline (speedup 1.0000x reference)
import functools

import jax
import jax.numpy as jnp
from jax import lax
from jax.experimental import pallas as pl
from jax.experimental.pallas import tpu as pltpu
from jax.experimental.pallas import tpu_sc as plsc

N_STAGES = 4
K = 8192
D = 32
TILE = 256


H = K // 2


def _stage_kernel(x_ref, et_ref, qv_refs, out_ref, et2b_ref, e2_ref, w_ref):
    @pl.when(pl.program_id(0) == 0)
    def _():
        et = et_ref[...]
        et2b_ref[...] = (et + et).astype(jnp.bfloat16)
        e2_ref[...] = jnp.sum(et * et, axis=0, keepdims=True)
        row = lax.broadcasted_iota(jnp.int32, (2 * H, 128), 0)
        col = lax.broadcasted_iota(jnp.int32, (2 * H, 128), 1)
        w = jnp.where(col == 0, row >> 6,
                      jnp.where(col == 1, row & 63,
                                jnp.where(col == 2, 1, 0)))
        w_ref[...] = w.astype(jnp.bfloat16)

    xt = x_ref[...]
    for qv_ref in qv_refs:
        xt = xt - qv_ref[:, :D]
    x2 = jnp.sum(xt * xt, axis=1, keepdims=True)
    s2 = jnp.dot(xt.astype(jnp.bfloat16), et2b_ref[...],
                 preferred_element_type=jnp.float32)
    dist = (x2 - s2) + e2_ref[...]
    d0 = dist[:, :H]
    d1 = dist[:, H:]
    m0 = jnp.min(d0, axis=1, keepdims=True)
    m1 = jnp.min(d1, axis=1, keepdims=True)
    take = m1 < m0.astype(jnp.bfloat16).astype(jnp.float32)
    inf = jnp.float32(jnp.inf)
    g0 = jnp.where(take, inf, m0)
    g1 = jnp.where(take, m1, inf)
    mask0 = (d0 == g0).astype(jnp.bfloat16)
    mask1 = (d1 == g1).astype(jnp.bfloat16)
    r = (jnp.dot(mask0, w_ref[:H], preferred_element_type=jnp.float32)
         + jnp.dot(mask1, w_ref[H:], preferred_element_type=jnp.float32))
    out_ref[0, :] = r[:, 0].astype(jnp.int32) * 64 + r[:, 1].astype(jnp.int32)

    @pl.when(jnp.max(r[:, 2]) > 1.5)
    def _():
        iota = lax.broadcasted_iota(jnp.int32, (TILE, H), 1)
        i0 = jnp.min(jnp.where(d0 == g0, iota, K), axis=1)
        i1 = jnp.min(jnp.where(d1 == g1, iota + H, K), axis=1)
        out_ref[0, :] = jnp.minimum(i0, i1)


def _tc_stage(n_prev, n, xf, et_q, qvs):
    body = lambda x_ref, et_ref, *rest: _stage_kernel(
        x_ref, et_ref, rest[:-4], rest[-4], rest[-3], rest[-2], rest[-1])
    out = pl.pallas_call(
        body,
        grid=(n // TILE,),
        in_specs=[pl.BlockSpec((TILE, D), lambda i: (i, 0)),
                  pl.BlockSpec((D, K), lambda i: (0, 0))]
        + [pl.BlockSpec((TILE, 128), lambda i: (i, 0))] * n_prev,
        out_specs=pl.BlockSpec((1, TILE), lambda i: (0, i)),
        out_shape=jax.ShapeDtypeStruct((1, n), jnp.int32),
        scratch_shapes=[pltpu.VMEM((D, K), jnp.bfloat16),
                        pltpu.VMEM((1, K), jnp.float32),
                        pltpu.VMEM((2 * H, 128), jnp.bfloat16)],
    )(xf, et_q, *qvs)
    return out[0]


def _make_sc_gather(n):
    info = plsc.get_sparse_core_info()
    nw = info.num_cores * info.num_subcores
    bpw = n // nw
    mesh = plsc.VectorSubcoreMesh(core_axis_name="c", subcore_axis_name="s")

    @functools.partial(
        pl.kernel, mesh=mesh,
        out_type=jax.ShapeDtypeStruct((n, 128), jnp.float32),
        scratch_types=[
            pltpu.VMEM((bpw,), jnp.int32),
            pltpu.VMEM((bpw, 128), jnp.float32),
            pltpu.SemaphoreType.DMA,
        ],
    )
    def gather_rows(table_hbm, idx_hbm, out_hbm, idx_v, rows_v, sem):
        wid = lax.axis_index("s") * info.num_cores + lax.axis_index("c")
        base = wid * bpw
        pltpu.sync_copy(idx_hbm.at[pl.ds(base, bpw)], idx_v)
        pltpu.async_copy(table_hbm.at[idx_v], rows_v, sem).wait()
        pltpu.sync_copy(rows_v, out_hbm.at[pl.ds(base, bpw)])

    return gather_rows


def kernel(x, embed):
    b, d, t = x.shape
    n = b * t
    xf = jnp.transpose(x, (0, 2, 1)).reshape(n, d)
    et = jnp.transpose(embed, (0, 2, 1))
    sc_gather = _make_sc_gather(n)
    embed_pad = jnp.pad(embed, ((0, 0), (0, 0), (0, 128 - D)))
    idxs = []
    qvs = []
    for q in range(N_STAGES):
        idx_q = _tc_stage(q, n, xf, et[q], qvs)
        idxs.append(idx_q)
        if q < N_STAGES - 1:
            qvs.append(sc_gather(embed_pad[q], idx_q))
    out = jnp.stack(idxs, axis=0)
    return jnp.transpose(out.reshape(N_STAGES, b, t), (1, 0, 2))


# speedup vs baseline: 1.1352x; 1.1352x over previous
import functools

import jax
import jax.numpy as jnp
from jax import lax
from jax.experimental import pallas as pl
from jax.experimental.pallas import tpu as pltpu
from jax.experimental.pallas import tpu_sc as plsc

N_STAGES = 4
K = 8192
D = 32
TILE = 256


H = K // 2


def _stage_kernel(x_ref, et_ref, qv_refs, out_ref, et2b_ref, e2_ref, w_ref):
    @pl.when(pl.program_id(0) == 0)
    def _():
        et = et_ref[...]
        et2b_ref[...] = (et + et).astype(jnp.bfloat16)
        e2_ref[...] = jnp.sum(et * et, axis=0, keepdims=True)
        row = lax.broadcasted_iota(jnp.int32, (2 * H, 128), 0)
        col = lax.broadcasted_iota(jnp.int32, (2 * H, 128), 1)
        w = jnp.where(col == 0, row >> 6,
                      jnp.where(col == 1, row & 63,
                                jnp.where(col == 2, 1, 0)))
        w_ref[...] = w.astype(jnp.bfloat16)

    xt = x_ref[...]
    for qv_ref in qv_refs:
        xt = xt - qv_ref[:, :D]
    x2 = jnp.sum(xt * xt, axis=1, keepdims=True)
    s2 = jnp.dot(xt.astype(jnp.bfloat16), et2b_ref[...],
                 preferred_element_type=jnp.float32)
    dist = (x2 - s2) + e2_ref[...]
    d0 = dist[:, :H]
    d1 = dist[:, H:]
    m0 = jnp.min(d0, axis=1, keepdims=True)
    m1 = jnp.min(d1, axis=1, keepdims=True)
    take = m1 < m0.astype(jnp.bfloat16).astype(jnp.float32)
    dwin = jnp.where(take, d1, d0)
    mwin = jnp.where(take, m1, m0)
    mask = (dwin == mwin).astype(jnp.bfloat16)
    r = jnp.dot(mask, w_ref[:H], preferred_element_type=jnp.float32)
    base = jnp.where(take[:, 0], H, 0)
    idx = r[:, 0].astype(jnp.int32) * 64 + r[:, 1].astype(jnp.int32)
    out_ref[0, :] = idx + base

    @pl.when(jnp.max(r[:, 2]) > 1.5)
    def _():
        iota = lax.broadcasted_iota(jnp.int32, (TILE, H), 1)
        iw = jnp.min(jnp.where(dwin == mwin, iota, K), axis=1)
        out_ref[0, :] = iw + base


def _tc_stage(n_prev, n, xf, et_q, qvs):
    body = lambda x_ref, et_ref, *rest: _stage_kernel(
        x_ref, et_ref, rest[:-4], rest[-4], rest[-3], rest[-2], rest[-1])
    out = pl.pallas_call(
        body,
        grid=(n // TILE,),
        in_specs=[pl.BlockSpec((TILE, D), lambda i: (i, 0)),
                  pl.BlockSpec((D, K), lambda i: (0, 0))]
        + [pl.BlockSpec((TILE, 128), lambda i: (i, 0))] * n_prev,
        out_specs=pl.BlockSpec((1, TILE), lambda i: (0, i)),
        out_shape=jax.ShapeDtypeStruct((1, n), jnp.int32),
        scratch_shapes=[pltpu.VMEM((D, K), jnp.bfloat16),
                        pltpu.VMEM((1, K), jnp.float32),
                        pltpu.VMEM((2 * H, 128), jnp.bfloat16)],
    )(xf, et_q, *qvs)
    return out[0]


def _make_sc_gather(n):
    info = plsc.get_sparse_core_info()
    nw = info.num_cores * info.num_subcores
    bpw = n // nw
    mesh = plsc.VectorSubcoreMesh(core_axis_name="c", subcore_axis_name="s")

    @functools.partial(
        pl.kernel, mesh=mesh,
        out_type=jax.ShapeDtypeStruct((n, 128), jnp.float32),
        scratch_types=[
            pltpu.VMEM((bpw,), jnp.int32),
            pltpu.VMEM((bpw, 128), jnp.float32),
            pltpu.SemaphoreType.DMA,
        ],
    )
    def gather_rows(table_hbm, idx_hbm, out_hbm, idx_v, rows_v, sem):
        wid = lax.axis_index("s") * info.num_cores + lax.axis_index("c")
        base = wid * bpw
        pltpu.sync_copy(idx_hbm.at[pl.ds(base, bpw)], idx_v)
        pltpu.async_copy(table_hbm.at[idx_v], rows_v, sem).wait()
        pltpu.sync_copy(rows_v, out_hbm.at[pl.ds(base, bpw)])

    return gather_rows


def kernel(x, embed):
    b, d, t = x.shape
    n = b * t
    xf = jnp.transpose(x, (0, 2, 1)).reshape(n, d)
    et = jnp.transpose(embed, (0, 2, 1))
    sc_gather = _make_sc_gather(n)
    embed_pad = jnp.pad(embed, ((0, 0), (0, 0), (0, 128 - D)))
    idxs = []
    qvs = []
    for q in range(N_STAGES):
        idx_q = _tc_stage(q, n, xf, et[q], qvs)
        idxs.append(idx_q)
        if q < N_STAGES - 1:
            qvs.append(sc_gather(embed_pad[q], idx_q))
    out = jnp.stack(idxs, axis=0)
    return jnp.transpose(out.reshape(N_STAGES, b, t), (1, 0, 2))
